# Initial kernel scaffold; baseline (speedup 1.0000x reference)
import jax
import jax.numpy as jnp
from jax.experimental import pallas as pl


def kernel(m_idx, p_idx, node_paths, node_signs, eta_bg, eta_meta, eta_pers):
    raise NotImplementedError("write your pallas kernel here")


# speedup vs baseline: 4.8595x; 4.8595x over previous
import functools

import jax
import jax.numpy as jnp
from jax import lax
from jax.experimental import pallas as pl
from jax.experimental.pallas import tpu as pltpu
from jax.experimental.pallas import tpu_sc as plsc


def _sc_logits(np3, rg3, m_idx, p_idx, eta_bg, meta_flat, pers_flat,
               *, NW, NC, n_sub, chunk, N):
    mesh = plsc.VectorSubcoreMesh(core_axis_name="c", subcore_axis_name="s")

    def body(np_hbm, rg_hbm, m_hbm, p_hbm, bg_hbm, meta_hbm, pers_hbm,
             out_hbm, npv, rv, cv, dv, av, bgv, mev, pev,
             sem_a, sem_b, sem_c):
        wid = lax.axis_index("s") * NC + lax.axis_index("c")

        pltpu.sync_copy(np_hbm.at[wid], npv)
        pltpu.sync_copy(rg_hbm.at[wid], rv)

        def fire_exp(j, carry):
            pltpu.make_async_copy(m_hbm.at[rv.at[j]], cv.at[j], sem_a).start()
            pltpu.make_async_copy(p_hbm.at[rv.at[j]], dv.at[j], sem_b).start()
            return carry

        lax.fori_loop(0, n_sub, fire_exp, 0)

        def drain_exp(j, carry):
            pltpu.make_async_copy(m_hbm.at[rv.at[j]], cv.at[j], sem_a).wait()
            pltpu.make_async_copy(p_hbm.at[rv.at[j]], dv.at[j], sem_b).wait()
            return carry

        lax.fori_loop(0, n_sub, drain_exp, 0)

        def build(i, carry):
            r = i // 8
            c = (i % 8) * 16
            n = npv[r, pl.ds(c, 16)]
            rv[r, pl.ds(c, 16)] = cv[r, pl.ds(c, 16)] * N + n
            av[r, pl.ds(c, 16)] = dv[r, pl.ds(c, 16)] * N + n
            return carry

        lax.fori_loop(0, chunk // 16, build, 0)

        def fire(j, carry):
            pltpu.make_async_copy(bg_hbm.at[npv.at[j]], bgv.at[j], sem_a).start()
            pltpu.make_async_copy(meta_hbm.at[rv.at[j]], mev.at[j], sem_b).start()
            pltpu.make_async_copy(pers_hbm.at[av.at[j]], pev.at[j], sem_c).start()
            return carry

        lax.fori_loop(0, n_sub, fire, 0)

        def drain(j, carry):
            pltpu.make_async_copy(bg_hbm.at[npv.at[j]], bgv.at[j], sem_a).wait()
            pltpu.make_async_copy(meta_hbm.at[rv.at[j]], mev.at[j], sem_b).wait()
            pltpu.make_async_copy(pers_hbm.at[av.at[j]], pev.at[j], sem_c).wait()
            return carry

        lax.fori_loop(0, n_sub, drain, 0)

        def sumi(i, carry):
            r = i // 8
            c = (i % 8) * 16
            bgv[r, pl.ds(c, 16)] = (bgv[r, pl.ds(c, 16)]
                                    + mev[r, pl.ds(c, 16)]
                                    + pev[r, pl.ds(c, 16)])
            return carry

        lax.fori_loop(0, chunk // 16, sumi, 0)

        pltpu.sync_copy(bgv, out_hbm.at[wid])

    run = pl.kernel(
        body,
        out_type=jax.ShapeDtypeStruct((NW, n_sub, 128), jnp.float32),
        mesh=mesh,
        scratch_types=[
            pltpu.VMEM((n_sub, 128), jnp.int32),
            pltpu.VMEM((n_sub, 128), jnp.int32),
            pltpu.VMEM((n_sub, 128), jnp.int32),
            pltpu.VMEM((n_sub, 128), jnp.int32),
            pltpu.VMEM((n_sub, 128), jnp.int32),
            pltpu.VMEM((n_sub, 128), jnp.float32),
            pltpu.VMEM((n_sub, 128), jnp.float32),
            pltpu.VMEM((n_sub, 128), jnp.float32),
            pltpu.SemaphoreType.DMA,
            pltpu.SemaphoreType.DMA,
            pltpu.SemaphoreType.DMA,
        ],
    )
    return run(np3, rg3, m_idx, p_idx, eta_bg, meta_flat, pers_flat)


def _tc_combine(logits, node_signs, node_paths):
    B, L = node_signs.shape
    blk = 1024
    grid = B // blk

    def body(lg_ref, sg_ref, np_ref, out_ref):
        x = sg_ref[...] * lg_ref[...]
        mask = (np_ref[...] != -1).astype(jnp.float32)
        lp = (jnp.minimum(x, 0.0) - jnp.log1p(jnp.exp(-jnp.abs(x)))) * mask
        out_ref[...] = jnp.sum(lp, axis=1, keepdims=True)

    out = pl.pallas_call(
        body,
        grid=(grid,),
        in_specs=[
            pl.BlockSpec((blk, L), lambda i: (i, 0)),
            pl.BlockSpec((blk, L), lambda i: (i, 0)),
            pl.BlockSpec((blk, L), lambda i: (i, 0)),
        ],
        out_specs=pl.BlockSpec((blk, 1), lambda i: (i, 0)),
        out_shape=jax.ShapeDtypeStruct((B, 1), jnp.float32),
    )(logits, node_signs, node_paths)
    return out.reshape(B)


def kernel(m_idx, p_idx, node_paths, node_signs, eta_bg, eta_meta, eta_pers):
    B, L = node_paths.shape
    M, N = eta_meta.shape
    info = plsc.get_sparse_core_info()
    NC, NS = info.num_cores, info.num_subcores
    NW = NC * NS
    chunk = (B * L) // NW
    n_sub = chunk // 128

    np3 = node_paths.reshape(NW, n_sub, 128)
    rg3 = (jnp.arange(B * L, dtype=jnp.int32) // L).reshape(NW, n_sub, 128)
    meta_flat = eta_meta.reshape(-1)
    pers_flat = eta_pers.reshape(-1)

    logits3 = _sc_logits(np3, rg3, m_idx, p_idx, eta_bg, meta_flat,
                         pers_flat, NW=NW, NC=NC, n_sub=n_sub,
                         chunk=chunk, N=N)
    logits = logits3.reshape(B, L)
    return _tc_combine(logits, node_signs, node_paths)
